# baseline (speedup 1.0000x reference)
import functools

import jax
import jax.numpy as jnp
from jax import lax
from jax.experimental import pallas as pl
from jax.experimental.pallas import tpu as pltpu
from jax.experimental.pallas import tpu_sc as plsc

B, T, H = 64, 2048, 256
NBINS, OUT, FILT, K = 256, 256, 256, 3
_EPS = 1e-5

_NC, _NS, _L = 2, 16, 16
_NW = _NC * _NS
_N = B * T
_RPW = _N // _NW
_CH = 128
_NCHUNK = _RPW // _CH


def _im2col3(x, pad_row):
    prv = jnp.concatenate([pad_row, x[:-1]], axis=0)
    nxt = jnp.concatenate([x[1:], pad_row], axis=0)
    return jnp.concatenate([prv, x, nxt], axis=1)


def _rowstats(h):
    mu = jnp.mean(h, axis=-1, keepdims=True)
    m2 = jnp.mean(h * h, axis=-1, keepdims=True)
    return mu, jax.lax.rsqrt(m2 - mu * mu + _EPS)


def _pred_kernel(x_ref, mask_ref, w1_ref, b1_ref,
                 p_w2, p_b2, p_pad, p_lwg, p_sc,
                 e_w2, e_b2, e_pad, e_lwg, e_sc,
                 ppred_ref, epred_ref):
    bf16 = jnp.bfloat16
    x = x_ref[0].astype(bf16)
    mask = mask_ref[0]
    zrow = jnp.zeros((1, H), bf16)

    xs = _im2col3(x, zrow)
    h12 = jnp.dot(xs, w1_ref[:, :], preferred_element_type=jnp.float32)
    h12 = jax.nn.relu(h12 + b1_ref[:, :])

    def head(h, w2, b2, pad, lwg, sc):
        mu, r = _rowstats(h)
        z = (h * r - mu * r).astype(bf16)
        zim = _im2col3(z, pad[:, :].astype(bf16))
        h2 = jnp.dot(zim, w2[:, :], preferred_element_type=jnp.float32)
        h2 = jax.nn.relu(h2 + b2[:, :])
        mu2, r2 = _rowstats(h2)
        s1 = jnp.sum(h2 * lwg[:, :], axis=-1, keepdims=True)
        pred = r2 * (s1 - mu2 * sc[0, 0]) + sc[0, 1]
        return jnp.where(mask > 0.0, 0.0, pred)

    ppred_ref[0] = head(h12[:, :FILT], p_w2, p_b2, p_pad, p_lwg, p_sc)
    epred_ref[0] = head(h12[:, FILT:], e_w2, e_b2, e_pad, e_lwg, e_sc)


def _bucketize_kernel(pt_ref, et_ref, pbins_ref, ebins_ref,
                      pidx_ref, eidx_ref):
    pcnt = jnp.sum((pt_ref[:, :] > pbins_ref[:, :]).astype(jnp.float32),
                   axis=-1, keepdims=True)
    ecnt = jnp.sum((et_ref[:, :] > ebins_ref[:, :]).astype(jnp.float32),
                   axis=-1, keepdims=True)
    pidx_ref[:, :] = pcnt.astype(jnp.int32)
    eidx_ref[:, :] = ecnt.astype(jnp.int32)


def _lookup_body(pidx_hbm, eidx_hbm, ptab_hbm, etab_hbm,
                 pout_hbm, eout_hbm,
                 idx_v, buf0, buf1, semg0, semg1, semw0, semw1):
    wid = lax.axis_index("s") * _NC + lax.axis_index("c")
    wbase = wid * _RPW
    last = _NCHUNK - 1

    for idx_hbm, tab_hbm, out_hbm in (
            (pidx_hbm, ptab_hbm, pout_hbm),
            (eidx_hbm, etab_hbm, eout_hbm)):
        pltpu.sync_copy(idx_hbm.at[pl.ds(wid * _NCHUNK, _NCHUNK)], idx_v)

        def startg(c, buf, sem):
            pltpu.async_copy(tab_hbm.at[idx_v.at[c]], buf, sem)

        def waitg(buf, sem):
            pltpu.make_async_copy(tab_hbm.at[idx_v.at[0]], buf, sem).wait()

        def startw(c, buf, sem):
            pltpu.async_copy(buf, out_hbm.at[pl.ds(wbase + c * _CH, _CH)],
                             sem)

        def waitw(c, buf, sem):
            pltpu.make_async_copy(
                buf, out_hbm.at[pl.ds(wbase + c * _CH, _CH)], sem).wait()

        startg(0, buf0, semg0)
        startg(1, buf1, semg1)

        def body(g, tok):
            c0 = 2 * g
            c1 = c0 + 1
            waitg(buf0, semg0)
            startw(c0, buf0, semw0)
            waitg(buf1, semg1)
            startw(c1, buf1, semw1)
            waitw(c0, buf0, semw0)
            startg(jnp.minimum(c0 + 2, last), buf0, semg0)
            waitw(c1, buf1, semw1)
            startg(jnp.minimum(c1 + 2, last), buf1, semg1)
            return tok

        lax.fori_loop(0, _NCHUNK // 2, body, 0)
        waitg(buf0, semg0)
        waitg(buf1, semg1)


def _row2(a):
    return a.reshape(1, -1)


def kernel(embeddings, src_mask, pitch_target, energy_target, pitch_bins,
           energy_bins, pitch_emb, energy_emb, p_params, e_params):
    f32, bf16 = jnp.float32, jnp.bfloat16
    mask_f = src_mask.astype(f32).reshape(B, T, 1)

    inf = jnp.full((1,), jnp.inf, f32)
    pbins = jnp.concatenate([pitch_bins, inf]).reshape(1, NBINS)
    ebins = jnp.concatenate([energy_bins, inf]).reshape(1, NBINS)

    whole = lambda shape: pl.BlockSpec(shape, lambda i: (0,) * len(shape))
    per_b3 = lambda shape: pl.BlockSpec(shape, lambda i: (i, 0, 0))

    _NROW = _N // 32
    rowblk = pl.BlockSpec((_NROW, 1), lambda i: (i, 0))
    pidx, eidx = pl.pallas_call(
        _bucketize_kernel,
        grid=(32,),
        in_specs=[rowblk, rowblk, whole((1, NBINS)), whole((1, NBINS))],
        out_specs=[rowblk, rowblk],
        out_shape=[jax.ShapeDtypeStruct((_N, 1), jnp.int32),
                   jax.ShapeDtypeStruct((_N, 1), jnp.int32)],
        compiler_params=pltpu.CompilerParams(
            dimension_semantics=("parallel",)),
    )(pitch_target.reshape(_N, 1), energy_target.reshape(_N, 1),
      pbins, ebins)

    sc_lookup = functools.partial(
        pl.kernel,
        out_type=[jax.ShapeDtypeStruct((_N, OUT), f32),
                  jax.ShapeDtypeStruct((_N, OUT), f32)],
        mesh=plsc.VectorSubcoreMesh(core_axis_name="c", subcore_axis_name="s",
                                    num_cores=_NC, num_subcores=_NS),
        scratch_types=[
            pltpu.VMEM((_NCHUNK, _CH), jnp.int32),
            pltpu.VMEM((_CH, OUT), f32),
            pltpu.VMEM((_CH, OUT), f32),
            pltpu.SemaphoreType.DMA,
            pltpu.SemaphoreType.DMA,
            pltpu.SemaphoreType.DMA,
            pltpu.SemaphoreType.DMA,
        ],
    )(_lookup_body)

    pembo, eembo = sc_lookup(pidx.reshape(_N // _CH, _CH),
                             eidx.reshape(_N // _CH, _CH),
                             pitch_emb, energy_emb)

    w1 = jnp.concatenate([p_params["conv1_w"].reshape(K * H, FILT),
                          e_params["conv1_w"].reshape(K * H, FILT)],
                         axis=1).astype(bf16)
    b1 = jnp.concatenate([p_params["conv1_b"], e_params["conv1_b"]])

    def head_params(p):
        g1, bt1 = p["ln1_g"], p["ln1_b"]
        w2g = (p["conv2_w"] * g1[None, :, None]).reshape(K * FILT, FILT)
        b2p = p["conv2_b"] + jnp.einsum("c,kcf->f", bt1, p["conv2_w"])
        pad = jnp.where(g1 != 0.0, -bt1 / jnp.where(g1 != 0.0, g1, 1.0), 0.0)
        lw = p["lin_w"][:, 0]
        lwg = lw * p["ln2_g"]
        sc = jnp.stack([jnp.sum(lwg),
                        jnp.sum(p["ln2_b"] * lw) + p["lin_b"][0]])
        return (w2g.astype(bf16), _row2(b2p), _row2(pad), _row2(lwg),
                sc.reshape(1, 2))

    in_specs = (
        [per_b3((1, T, H)), per_b3((1, T, 1))]
        + [whole((K * H, 2 * FILT)), whole((1, 2 * FILT))]
        + [whole((K * FILT, FILT)), whole((1, FILT)), whole((1, FILT)),
           whole((1, FILT)), whole((1, 2))] * 2
    )
    out_specs = [per_b3((1, T, 1)), per_b3((1, T, 1))]
    out_shape = [jax.ShapeDtypeStruct((B, T, 1), f32),
                 jax.ShapeDtypeStruct((B, T, 1), f32)]

    ppred, epred = pl.pallas_call(
        _pred_kernel,
        grid=(B,),
        in_specs=in_specs,
        out_specs=out_specs,
        out_shape=out_shape,
        compiler_params=pltpu.CompilerParams(
            dimension_semantics=("parallel",),
            vmem_limit_bytes=120 * 1024 * 1024),
    )(embeddings, mask_f, w1, _row2(b1),
      *head_params(p_params), *head_params(e_params))

    return (ppred.reshape(B, T), pembo.reshape(B, T, OUT),
            epred.reshape(B, T), eembo.reshape(B, T, OUT))


# speedup vs baseline: 1.2500x; 1.2500x over previous
import functools

import jax
import jax.numpy as jnp
from jax import lax
from jax.experimental import pallas as pl
from jax.experimental.pallas import tpu as pltpu
from jax.experimental.pallas import tpu_sc as plsc

B, T, H = 64, 2048, 256
NBINS, OUT, FILT, K = 256, 256, 256, 3
_EPS = 1e-5

_NC, _NS, _L = 2, 16, 16
_NW = _NC * _NS
_N = B * T
_RPW = _N // _NW
_CH = 128
_NCHUNK = _RPW // _CH


def _im2col3(x, pad_row):
    prv = jnp.concatenate([pad_row, x[:-1]], axis=0)
    nxt = jnp.concatenate([x[1:], pad_row], axis=0)
    return jnp.concatenate([prv, x, nxt], axis=1)


def _rowstats(h):
    mu = jnp.mean(h, axis=-1, keepdims=True)
    m2 = jnp.mean(h * h, axis=-1, keepdims=True)
    return mu, jax.lax.rsqrt(m2 - mu * mu + _EPS)


def _pred_kernel(x_ref, mask_ref, et_ref, elo_ref, ehi_ref, eemb_ref,
                 w1_ref, b1_ref,
                 p_w2, p_b2, p_pad, p_lwg, p_sc,
                 e_w2, e_b2, e_pad, e_lwg, e_sc,
                 ppred_ref, epred_ref, eembo_ref):
    bf16 = jnp.bfloat16
    x = x_ref[0].astype(bf16)
    mask = mask_ref[0]
    zrow = jnp.zeros((1, H), bf16)

    xs = _im2col3(x, zrow)
    h12 = jnp.dot(xs, w1_ref[:, :], preferred_element_type=jnp.float32)
    h12 = jax.nn.relu(h12 + b1_ref[:, :])

    def head(h, w2, b2, pad, lwg, sc):
        mu, r = _rowstats(h)
        z = (h * r - mu * r).astype(bf16)
        zim = _im2col3(z, pad[:, :].astype(bf16))
        h2 = jnp.dot(zim, w2[:, :], preferred_element_type=jnp.float32)
        h2 = jax.nn.relu(h2 + b2[:, :])
        mu2, r2 = _rowstats(h2)
        s1 = jnp.sum(h2 * lwg[:, :], axis=-1, keepdims=True)
        pred = r2 * (s1 - mu2 * sc[0, 0]) + sc[0, 1]
        return jnp.where(mask > 0.0, 0.0, pred)

    ppred_ref[0] = head(h12[:, :FILT], p_w2, p_b2, p_pad, p_lwg, p_sc)
    epred_ref[0] = head(h12[:, FILT:], e_w2, e_b2, e_pad, e_lwg, e_sc)

    ev = et_ref[0]
    oh = ((ev > elo_ref[:, :]) & (ev <= ehi_ref[:, :])).astype(jnp.bfloat16)
    eembo_ref[0] = jnp.dot(oh, eemb_ref[:, :],
                           preferred_element_type=jnp.float32)


def _bucketize_kernel(pt_ref, pbins_ref, pidx_ref):
    pcnt = jnp.sum((pt_ref[:, :] > pbins_ref[:, :]).astype(jnp.float32),
                   axis=-1, keepdims=True)
    pidx_ref[:, :] = pcnt.astype(jnp.int32)


def _lookup_body(pidx_hbm, ptab_hbm, pout_hbm,
                 idx_v, buf0, buf1, semg0, semg1, semw0, semw1):
    wid = lax.axis_index("s") * _NC + lax.axis_index("c")
    wbase = wid * _RPW
    last = _NCHUNK - 1

    for idx_hbm, tab_hbm, out_hbm in ((pidx_hbm, ptab_hbm, pout_hbm),):
        pltpu.sync_copy(idx_hbm.at[pl.ds(wid * _NCHUNK, _NCHUNK)], idx_v)

        def startg(c, buf, sem):
            pltpu.async_copy(tab_hbm.at[idx_v.at[c]], buf, sem)

        def waitg(buf, sem):
            pltpu.make_async_copy(tab_hbm.at[idx_v.at[0]], buf, sem).wait()

        def startw(c, buf, sem):
            pltpu.async_copy(buf, out_hbm.at[pl.ds(wbase + c * _CH, _CH)],
                             sem)

        def waitw(c, buf, sem):
            pltpu.make_async_copy(
                buf, out_hbm.at[pl.ds(wbase + c * _CH, _CH)], sem).wait()

        startg(0, buf0, semg0)
        startg(1, buf1, semg1)

        def body(g, tok):
            c0 = 2 * g
            c1 = c0 + 1
            waitg(buf0, semg0)
            startw(c0, buf0, semw0)
            waitg(buf1, semg1)
            startw(c1, buf1, semw1)
            waitw(c0, buf0, semw0)
            startg(jnp.minimum(c0 + 2, last), buf0, semg0)
            waitw(c1, buf1, semw1)
            startg(jnp.minimum(c1 + 2, last), buf1, semg1)
            return tok

        lax.fori_loop(0, _NCHUNK // 2, body, 0)
        waitg(buf0, semg0)
        waitg(buf1, semg1)


def _row2(a):
    return a.reshape(1, -1)


def kernel(embeddings, src_mask, pitch_target, energy_target, pitch_bins,
           energy_bins, pitch_emb, energy_emb, p_params, e_params):
    f32, bf16 = jnp.float32, jnp.bfloat16
    mask_f = src_mask.astype(f32).reshape(B, T, 1)

    inf = jnp.full((1,), jnp.inf, f32)
    pbins = jnp.concatenate([pitch_bins, inf]).reshape(1, NBINS)
    ebins = jnp.concatenate([energy_bins, inf]).reshape(1, NBINS)

    whole = lambda shape: pl.BlockSpec(shape, lambda i: (0,) * len(shape))
    per_b3 = lambda shape: pl.BlockSpec(shape, lambda i: (i, 0, 0))

    _NROW = _N // 32
    rowblk = pl.BlockSpec((_NROW, 1), lambda i: (i, 0))
    pidx, = pl.pallas_call(
        _bucketize_kernel,
        grid=(32,),
        in_specs=[rowblk, whole((1, NBINS))],
        out_specs=[rowblk],
        out_shape=[jax.ShapeDtypeStruct((_N, 1), jnp.int32)],
        compiler_params=pltpu.CompilerParams(
            dimension_semantics=("parallel",)),
    )(pitch_target.reshape(_N, 1), pbins)

    sc_lookup = functools.partial(
        pl.kernel,
        out_type=[jax.ShapeDtypeStruct((_N, OUT), f32)],
        mesh=plsc.VectorSubcoreMesh(core_axis_name="c", subcore_axis_name="s",
                                    num_cores=_NC, num_subcores=_NS),
        scratch_types=[
            pltpu.VMEM((_NCHUNK, _CH), jnp.int32),
            pltpu.VMEM((_CH, OUT), f32),
            pltpu.VMEM((_CH, OUT), f32),
            pltpu.SemaphoreType.DMA,
            pltpu.SemaphoreType.DMA,
            pltpu.SemaphoreType.DMA,
            pltpu.SemaphoreType.DMA,
        ],
    )(_lookup_body)

    pembo, = sc_lookup(pidx.reshape(_N // _CH, _CH), pitch_emb)

    w1 = jnp.concatenate([p_params["conv1_w"].reshape(K * H, FILT),
                          e_params["conv1_w"].reshape(K * H, FILT)],
                         axis=1).astype(bf16)
    b1 = jnp.concatenate([p_params["conv1_b"], e_params["conv1_b"]])

    def head_params(p):
        g1, bt1 = p["ln1_g"], p["ln1_b"]
        w2g = (p["conv2_w"] * g1[None, :, None]).reshape(K * FILT, FILT)
        b2p = p["conv2_b"] + jnp.einsum("c,kcf->f", bt1, p["conv2_w"])
        pad = jnp.where(g1 != 0.0, -bt1 / jnp.where(g1 != 0.0, g1, 1.0), 0.0)
        lw = p["lin_w"][:, 0]
        lwg = lw * p["ln2_g"]
        sc = jnp.stack([jnp.sum(lwg),
                        jnp.sum(p["ln2_b"] * lw) + p["lin_b"][0]])
        return (w2g.astype(bf16), _row2(b2p), _row2(pad), _row2(lwg),
                sc.reshape(1, 2))

    in_specs = (
        [per_b3((1, T, H)), per_b3((1, T, 1)), per_b3((1, T, 1)),
         whole((1, NBINS)), whole((1, NBINS)), whole((NBINS, OUT))]
        + [whole((K * H, 2 * FILT)), whole((1, 2 * FILT))]
        + [whole((K * FILT, FILT)), whole((1, FILT)), whole((1, FILT)),
           whole((1, FILT)), whole((1, 2))] * 2
    )
    out_specs = [per_b3((1, T, 1)), per_b3((1, T, 1)), per_b3((1, T, OUT))]
    out_shape = [jax.ShapeDtypeStruct((B, T, 1), f32),
                 jax.ShapeDtypeStruct((B, T, 1), f32),
                 jax.ShapeDtypeStruct((B, T, OUT), f32)]

    elo = jnp.concatenate([-inf, energy_bins]).reshape(1, NBINS)
    ppred, epred, eembo = pl.pallas_call(
        _pred_kernel,
        grid=(B,),
        in_specs=in_specs,
        out_specs=out_specs,
        out_shape=out_shape,
        compiler_params=pltpu.CompilerParams(
            dimension_semantics=("parallel",),
            vmem_limit_bytes=120 * 1024 * 1024),
    )(embeddings, mask_f, energy_target.reshape(B, T, 1), elo, ebins,
      energy_emb.astype(bf16), w1, _row2(b1),
      *head_params(p_params), *head_params(e_params))

    return (ppred.reshape(B, T), pembo.reshape(B, T, OUT),
            epred.reshape(B, T), eembo.reshape(B, T, OUT))


# speedup vs baseline: 1.5937x; 1.2750x over previous
import jax
import jax.numpy as jnp
from jax.experimental import pallas as pl
from jax.experimental.pallas import tpu as pltpu

B, T, H = 64, 2048, 256
NBINS, OUT, FILT, K = 256, 256, 256, 3
_EPS = 1e-5
_BB = 2


def _im2col3(x, pad_row):
    prv = jnp.concatenate([pad_row, x[:-1]], axis=0)
    nxt = jnp.concatenate([x[1:], pad_row], axis=0)
    return jnp.concatenate([prv, x, nxt], axis=1)


def _rowstats(h):
    mu = jnp.mean(h, axis=-1, keepdims=True)
    m2 = jnp.mean(h * h, axis=-1, keepdims=True)
    return mu, jax.lax.rsqrt(m2 - mu * mu + _EPS)


def _onehot_lookup(v_col, lo_row, hi_row, emb_ref):
    oh = ((v_col > lo_row) & (v_col <= hi_row)).astype(jnp.bfloat16)
    return jnp.dot(oh, emb_ref[:, :], preferred_element_type=jnp.float32)


def _va_kernel(x_ref, mask_ref, pt_ref, et_ref,
               plo_ref, phi_ref, elo_ref, ehi_ref,
               pemb_ref, eemb_ref,
               w1_ref, b1_ref,
               p_w2, p_b2, p_pad, p_lwg, p_sc,
               e_w2, e_b2, e_pad, e_lwg, e_sc,
               ppred_ref, pembo_ref, epred_ref, eembo_ref):
    bf16 = jnp.bfloat16
    zrow = jnp.zeros((1, H), bf16)

    for j in range(_BB):
        x = x_ref[j].astype(bf16)
        mask = mask_ref[j]

        xs = _im2col3(x, zrow)
        h12 = jnp.dot(xs, w1_ref[:, :], preferred_element_type=jnp.float32)
        h12 = jax.nn.relu(h12 + b1_ref[:, :])

        def head(h, w2, b2, pad, lwg, sc):
            mu, r = _rowstats(h)
            z = (h * r - mu * r).astype(bf16)
            zim = _im2col3(z, pad[:, :].astype(bf16))
            h2 = jnp.dot(zim, w2[:, :], preferred_element_type=jnp.float32)
            h2 = jax.nn.relu(h2 + b2[:, :])
            mu2, r2 = _rowstats(h2)
            s1 = jnp.sum(h2 * lwg[:, :], axis=-1, keepdims=True)
            pred = r2 * (s1 - mu2 * sc[0, 0]) + sc[0, 1]
            return jnp.where(mask > 0.0, 0.0, pred)

        ppred_ref[j] = head(h12[:, :FILT], p_w2, p_b2, p_pad, p_lwg, p_sc)
        epred_ref[j] = head(h12[:, FILT:], e_w2, e_b2, e_pad, e_lwg, e_sc)

        pembo_ref[j] = _onehot_lookup(pt_ref[j], plo_ref[:, :],
                                      phi_ref[:, :], pemb_ref)
        eembo_ref[j] = _onehot_lookup(et_ref[j], elo_ref[:, :],
                                      ehi_ref[:, :], eemb_ref)


def _row2(a):
    return a.reshape(1, -1)


def kernel(embeddings, src_mask, pitch_target, energy_target, pitch_bins,
           energy_bins, pitch_emb, energy_emb, p_params, e_params):
    f32, bf16 = jnp.float32, jnp.bfloat16
    mask_f = src_mask.astype(f32).reshape(B, T, 1)
    pt = pitch_target.reshape(B, T, 1)
    et = energy_target.reshape(B, T, 1)

    inf = jnp.full((1,), jnp.inf, f32)
    plo = jnp.concatenate([-inf, pitch_bins]).reshape(1, NBINS)
    phi = jnp.concatenate([pitch_bins, inf]).reshape(1, NBINS)
    elo = jnp.concatenate([-inf, energy_bins]).reshape(1, NBINS)
    ehi = jnp.concatenate([energy_bins, inf]).reshape(1, NBINS)

    w1 = jnp.concatenate([p_params["conv1_w"].reshape(K * H, FILT),
                          e_params["conv1_w"].reshape(K * H, FILT)],
                         axis=1).astype(bf16)
    b1 = jnp.concatenate([p_params["conv1_b"], e_params["conv1_b"]])

    def head_params(p):
        g1, bt1 = p["ln1_g"], p["ln1_b"]
        w2g = (p["conv2_w"] * g1[None, :, None]).reshape(K * FILT, FILT)
        b2p = p["conv2_b"] + jnp.einsum("c,kcf->f", bt1, p["conv2_w"])
        pad = jnp.where(g1 != 0.0, -bt1 / jnp.where(g1 != 0.0, g1, 1.0), 0.0)
        lw = p["lin_w"][:, 0]
        lwg = lw * p["ln2_g"]
        sc = jnp.stack([jnp.sum(lwg),
                        jnp.sum(p["ln2_b"] * lw) + p["lin_b"][0]])
        return (w2g.astype(bf16), _row2(b2p), _row2(pad), _row2(lwg),
                sc.reshape(1, 2))

    whole = lambda shape: pl.BlockSpec(shape, lambda i: (0,) * len(shape))
    per_b3 = lambda shape: pl.BlockSpec(shape, lambda i: (i, 0, 0))

    in_specs = (
        [per_b3((_BB, T, H)), per_b3((_BB, T, 1)), per_b3((_BB, T, 1)),
         per_b3((_BB, T, 1))]
        + [whole((1, NBINS))] * 4
        + [whole((NBINS, OUT))] * 2
        + [whole((K * H, 2 * FILT)), whole((1, 2 * FILT))]
        + [whole((K * FILT, FILT)), whole((1, FILT)), whole((1, FILT)),
           whole((1, FILT)), whole((1, 2))] * 2
    )
    out_specs = [per_b3((_BB, T, 1)), per_b3((_BB, T, OUT)),
                 per_b3((_BB, T, 1)), per_b3((_BB, T, OUT))]
    out_shape = [jax.ShapeDtypeStruct((B, T, 1), f32),
                 jax.ShapeDtypeStruct((B, T, OUT), f32),
                 jax.ShapeDtypeStruct((B, T, 1), f32),
                 jax.ShapeDtypeStruct((B, T, OUT), f32)]

    ppred, pembo, epred, eembo = pl.pallas_call(
        _va_kernel,
        grid=(B // _BB,),
        in_specs=in_specs,
        out_specs=out_specs,
        out_shape=out_shape,
        compiler_params=pltpu.CompilerParams(
            dimension_semantics=("parallel",),
            vmem_limit_bytes=120 * 1024 * 1024),
    )(embeddings, mask_f, pt, et, plo, phi, elo, ehi,
      pitch_emb.astype(bf16), energy_emb.astype(bf16),
      w1, _row2(b1), *head_params(p_params), *head_params(e_params))

    return (ppred.reshape(B, T), pembo, epred.reshape(B, T), eembo)
